# baseline (speedup 1.0000x reference)
import functools

import jax
import jax.numpy as jnp
from jax import lax
from jax.experimental import pallas as pl
from jax.experimental.pallas import tpu as pltpu
from jax.experimental.pallas import tpu_sc as plsc

IDX_MINOR = 128


@functools.cache
def _make_sc_gather(B: int, V: int, D: int):
    info = plsc.get_sparse_core_info()
    num_workers = info.num_cores * info.num_subcores
    rows_per_w = B // num_workers
    idxrows_per_w = rows_per_w // IDX_MINOR
    gathers_per_chunk = 10
    chunk = gathers_per_chunk * IDX_MINOR
    nchunk = rows_per_w // chunk
    assert chunk * nchunk == rows_per_w and nchunk % 2 == 0

    mesh = plsc.VectorSubcoreMesh(core_axis_name="c", subcore_axis_name="s")

    @functools.partial(
        pl.kernel,
        out_type=jax.ShapeDtypeStruct((B, D), jnp.float32),
        mesh=mesh,
        scratch_types=[
            pltpu.VMEM((idxrows_per_w, IDX_MINOR), jnp.int32),
            pltpu.VMEM((2, chunk, D), jnp.float32),
            pltpu.SemaphoreType.DMA,
            pltpu.SemaphoreType.DMA,
        ],
        compiler_params=pltpu.CompilerParams(use_tc_tiling_on_sc=False),
    )
    def gather_kernel(idx_hbm, table_hbm, out_hbm, idx_v, rows_v, gsem, wsem):
        wid = lax.axis_index("s") * info.num_cores + lax.axis_index("c")
        pltpu.sync_copy(
            idx_hbm.at[pl.ds(wid * idxrows_per_w, idxrows_per_w)], idx_v
        )
        out_base = wid * rows_per_w

        @pl.loop(0, nchunk, step=2)
        def _pair(c0):
            for b in range(2):
                c = c0 + b

                @pl.when(c0 > 0)
                def _():
                    pltpu.make_async_copy(
                        out_hbm.at[pl.ds(out_base, chunk)], rows_v.at[b], wsem
                    ).wait()

                descs = []
                for j in range(gathers_per_chunk):
                    irow = c * gathers_per_chunk + j
                    descs.append(
                        pltpu.async_copy(
                            table_hbm.at[idx_v.at[irow]],
                            rows_v.at[b].at[pl.ds(j * IDX_MINOR, IDX_MINOR)],
                            gsem,
                        )
                    )
                for d in descs:
                    d.wait()
                pltpu.async_copy(
                    rows_v.at[b], out_hbm.at[pl.ds(out_base + c * chunk, chunk)], wsem
                )

        for b in range(2):
            pltpu.make_async_copy(
                out_hbm.at[pl.ds(out_base, chunk)], rows_v.at[b], wsem
            ).wait()

    return gather_kernel


@jax.jit
def kernel(x, embeddings):
    N_, L_ = x.shape
    V, D = embeddings.shape
    B = N_ * L_
    idx = x.reshape(B // IDX_MINOR, IDX_MINOR).astype(jnp.int32)
    out = _make_sc_gather(B, V, D)(idx, embeddings)
    return out.reshape(N_, L_, D)


# speedup vs baseline: 1.3638x; 1.3638x over previous
import functools

import jax
import jax.numpy as jnp
from jax import lax
from jax.experimental import pallas as pl
from jax.experimental.pallas import tpu as pltpu
from jax.experimental.pallas import tpu_sc as plsc

IDX_MINOR = 128


@functools.cache
def _make_sc_lookup(N: int, L: int, V: int, D: int):
    info = plsc.get_sparse_core_info()
    NC = info.num_cores
    num_workers = NC * info.num_subcores
    B = N * L
    DT = D // 8
    NT = N // IDX_MINOR
    NTG = 4
    n_tasks = L * (NT // NTG)
    tasks_per_w = n_tasks // num_workers
    rows_per_task = NTG * IDX_MINOR
    idxrows_per_w = tasks_per_w * NTG

    mesh = plsc.VectorSubcoreMesh(core_axis_name="c", subcore_axis_name="s")

    @functools.partial(
        pl.kernel,
        out_type=jax.ShapeDtypeStruct((L, DT, NT, 8, IDX_MINOR), jnp.float32),
        mesh=mesh,
        scratch_types=[
            pltpu.VMEM((idxrows_per_w, IDX_MINOR), jnp.int32),
            pltpu.VMEM((rows_per_task, D), jnp.float32),
            pltpu.VMEM((DT, NTG, 8, IDX_MINOR), jnp.float32),
            pltpu.SemaphoreType.DMA,
        ],
        compiler_params=pltpu.CompilerParams(
            use_tc_tiling_on_sc=False, needs_layout_passes=False
        ),
    )
    def lookup_kernel(idx_hbm, table_hbm, out_hbm, idx_v, rows_v, tr_v, sem):
        wid = lax.axis_index("s") * NC + lax.axis_index("c")
        pltpu.sync_copy(
            idx_hbm.at[pl.ds(wid * idxrows_per_w, idxrows_per_w)], idx_v
        )
        task0 = wid * tasks_per_w

        @pl.loop(0, tasks_per_w)
        def _task(tl):
            task = task0 + tl
            l = task // (NT // NTG)
            ntg = task % (NT // NTG)

            descs = []
            for j in range(NTG):
                descs.append(
                    pltpu.async_copy(
                        table_hbm.at[idx_v.at[tl * NTG + j]],
                        rows_v.at[pl.ds(j * IDX_MINOR, IDX_MINOR)],
                        sem,
                    )
                )
            for d in descs:
                d.wait()

            @pl.loop(0, D)
            def _d(d):
                dt = d // 8
                ds_ = d % 8
                d_idx = jnp.full((16,), 0, jnp.int32) + d
                for j in range(NTG):
                    for k in range(IDX_MINOR // 16):
                        n_idx = lax.iota(jnp.int32, 16) + (j * IDX_MINOR + k * 16)
                        vals = plsc.load_gather(rows_v, [n_idx, d_idx])
                        tr_v[dt, j, ds_, pl.ds(k * 16, 16)] = vals

            pltpu.sync_copy(
                tr_v, out_hbm.at[l, :, pl.ds(ntg * NTG, NTG)]
            )

    return lookup_kernel


@jax.jit
def kernel(x, embeddings):
    N_, L_ = x.shape
    V, D = embeddings.shape
    B = N_ * L_
    idx = x.T.reshape(B // IDX_MINOR, IDX_MINOR).astype(jnp.int32)
    out5d = _make_sc_lookup(N_, L_, V, D)(idx, embeddings)
    return out5d.transpose(2, 4, 0, 1, 3).reshape(N_, L_, D)
